# Initial kernel scaffold; baseline (speedup 1.0000x reference)
import jax
import jax.numpy as jnp
from jax.experimental import pallas as pl


def kernel(vertices, img_feats, proj_mat):
    raise NotImplementedError("write your pallas kernel here")


# speedup vs baseline: 7.9396x; 7.9396x over previous
import functools

import jax
import jax.numpy as jnp
from jax import lax
from jax.experimental import pallas as pl
from jax.experimental.pallas import tpu as pltpu
from jax.experimental.pallas import tpu_sc as plsc

B, N, C = 16, 8192, 256
HW = 56
LEVEL_SIZES = (56, 28, 14, 7)
NLVL = 4
FX, FY, CX, CY = 250.0, 250.0, 112.0, 112.0
IMG_H, IMG_W = 224.0, 224.0
OUTD = 3 + NLVL * C
PTS = B * N

NC, NS, L = 2, 16, 16
NW = NC * NS
PPW = PTS // NW
CHUNK = 16
NCHUNK = PPW // CHUNK


def _sc_project(verts_t, table):
    mesh = plsc.VectorSubcoreMesh(
        core_axis_name="c", subcore_axis_name="s",
        num_cores=NC, num_subcores=NS)

    @functools.partial(
        pl.kernel,
        out_type=jax.ShapeDtypeStruct((PTS * OUTD,), jnp.float32),
        mesh=mesh,
        compiler_params=pltpu.CompilerParams(needs_layout_passes=False),
        scratch_types=[
            pltpu.VMEM((3, PPW), jnp.float32),
            pltpu.VMEM((4 * L,), jnp.int32),
            pltpu.VMEM((4 * L, C), jnp.float32),
            pltpu.VMEM((CHUNK * OUTD,), jnp.float32),
            pltpu.SemaphoreType.DMA,
        ],
    )
    def k(verts_hbm, table_hbm, out_hbm, verts_v, idx_v, rows_v,
          outb_v, sem):
        wid = lax.axis_index("s") * NC + lax.axis_index("c")
        base = wid * PPW
        bidx = base // N
        pltpu.sync_copy(verts_hbm.at[:, pl.ds(base, PPW)], verts_v)

        lane = lax.iota(jnp.int32, L)

        def chunk_body(ci, _):
            off = ci * CHUNK
            xv = verts_v[0, pl.ds(off, L)]
            yv = verts_v[1, pl.ds(off, L)]
            zv = verts_v[2, pl.ds(off, L)]
            h = FY * (yv / zv) + CY
            w = FX * (xv / (-zv)) + CX

            rowoff = lane * OUTD
            plsc.store_scatter(outb_v, [rowoff + 0], xv)
            plsc.store_scatter(outb_v, [rowoff + 1], yv)
            plsc.store_scatter(outb_v, [rowoff + 2], zv)

            for lvl, size in enumerate(LEVEL_SIZES):
                x = jnp.clip(h * (size / IMG_H), 0.0, size - 1.0)
                y = jnp.clip(w * (size / IMG_W), 0.0, size - 1.0)
                x1i = x.astype(jnp.int32)
                x1f = x1i.astype(jnp.float32)
                gx = (x > x1f).astype(jnp.int32)
                x2i = x1i + gx
                x2f = x2i.astype(jnp.float32)
                y1i = y.astype(jnp.int32)
                y1f = y1i.astype(jnp.float32)
                gy = (y > y1f).astype(jnp.int32)
                y2i = y1i + gy
                y2f = y2i.astype(jnp.float32)

                w11v = (x2f - x) * (y2f - y)
                w21v = (x - x1f) * (y2f - y)
                w12v = (x2f - x) * (y - y1f)
                w22v = (x - x1f) * (y - y1f)

                rowbase = (lvl * B + bidx) * (HW * HW)
                r1 = rowbase + x1i * HW
                r2 = rowbase + x2i * HW
                idx_v[pl.ds(0 * L, L)] = r1 + y1i
                idx_v[pl.ds(1 * L, L)] = r2 + y1i
                idx_v[pl.ds(2 * L, L)] = r1 + y2i
                idx_v[pl.ds(3 * L, L)] = r2 + y2i

                pltpu.async_copy(table_hbm.at[idx_v], rows_v, sem).wait()

                col0 = 3 + lvl * C

                def blend_p(p, _):
                    pfull = jnp.full((L,), p, jnp.int32)
                    w11 = w11v.at[pfull].get(mode="promise_in_bounds")
                    w21 = w21v.at[pfull].get(mode="promise_in_bounds")
                    w12 = w12v.at[pfull].get(mode="promise_in_bounds")
                    w22 = w22v.at[pfull].get(mode="promise_in_bounds")
                    for cc in range(C // L):
                        sl = pl.ds(cc * L, L)
                        acc = (w11 * rows_v[p, sl]
                               + w21 * rows_v[L + p, sl]
                               + w12 * rows_v[2 * L + p, sl]
                               + w22 * rows_v[3 * L + p, sl])
                        outb_v[pl.ds(p * OUTD + col0 + cc * L, L)] = acc
                    return 0

                lax.fori_loop(0, CHUNK, blend_p, 0)

            pltpu.sync_copy(outb_v,
                            out_hbm.at[pl.ds((base + off) * OUTD, CHUNK * OUTD)])
            return 0

        lax.fori_loop(0, NCHUNK, chunk_body, 0)

    return k(verts_t, table)


def kernel(vertices, img_feats, proj_mat):
    del proj_mat
    table = jnp.transpose(img_feats, (0, 1, 3, 4, 2)).reshape(NLVL * B * HW * HW, C)
    verts_t = jnp.transpose(vertices.reshape(PTS, 3), (1, 0))
    out = _sc_project(verts_t, table)
    return out.reshape(B, N, OUTD)


# speedup vs baseline: 10.6612x; 1.3428x over previous
import functools

import jax
import jax.numpy as jnp
from jax import lax
from jax.experimental import pallas as pl
from jax.experimental.pallas import tpu as pltpu
from jax.experimental.pallas import tpu_sc as plsc

B, N, C = 16, 8192, 256
HW = 56
LEVEL_SIZES = (56, 28, 14, 7)
NLVL = 4
FX, FY, CX, CY = 250.0, 250.0, 112.0, 112.0
IMG_H, IMG_W = 224.0, 224.0
OUTD = 3 + NLVL * C
PTS = B * N

NC, NS, L = 2, 16, 16
NW = NC * NS
PPW = PTS // NW
CHUNK = 16
NCHUNK = PPW // CHUNK
OUTB = CHUNK * OUTD


def _sc_project(verts_t, table):
    mesh = plsc.VectorSubcoreMesh(
        core_axis_name="c", subcore_axis_name="s",
        num_cores=NC, num_subcores=NS)

    @functools.partial(
        pl.kernel,
        out_type=jax.ShapeDtypeStruct((PTS * OUTD,), jnp.float32),
        mesh=mesh,
        compiler_params=pltpu.CompilerParams(needs_layout_passes=False),
        scratch_types=[
            pltpu.VMEM((3, PPW), jnp.float32),
            pltpu.VMEM((4 * L,), jnp.int32),
            pltpu.VMEM((4 * L,), jnp.int32),
            pltpu.VMEM((4 * L,), jnp.int32),
            pltpu.VMEM((4 * L,), jnp.int32),
            pltpu.VMEM((4 * L, C), jnp.float32),
            pltpu.VMEM((4 * L, C), jnp.float32),
            pltpu.VMEM((2 * OUTB,), jnp.float32),
            pltpu.SemaphoreType.DMA,
            pltpu.SemaphoreType.DMA,
            pltpu.SemaphoreType.DMA,
        ],
    )
    def k(verts_hbm, table_hbm, out_hbm, verts_v, idx0, idx1, idx2, idx3,
          rows_a, rows_b, outb_v, sem_a, sem_b, sem_out):
        wid = lax.axis_index("s") * NC + lax.axis_index("c")
        base = wid * PPW
        bidx = base // N
        pltpu.sync_copy(verts_hbm.at[:, pl.ds(base, PPW)], verts_v)

        lane = lax.iota(jnp.int32, L)
        idx_refs = (idx0, idx1, idx2, idx3)
        row_bufs = (rows_a, rows_b)
        sems = (sem_a, sem_b)

        def chunk_body(ci, _):
            off = ci * CHUNK
            poff = lax.rem(ci, 2) * OUTB
            xv = verts_v[0, pl.ds(off, L)]
            yv = verts_v[1, pl.ds(off, L)]
            zv = verts_v[2, pl.ds(off, L)]
            h = FY * (yv / zv) + CY
            w = FX * (xv / (-zv)) + CX

            rowoff = poff + lane * OUTD
            plsc.store_scatter(outb_v, [rowoff + 0], xv)
            plsc.store_scatter(outb_v, [rowoff + 1], yv)
            plsc.store_scatter(outb_v, [rowoff + 2], zv)

            wts = []
            for lvl, size in enumerate(LEVEL_SIZES):
                x = jnp.clip(h * (size / IMG_H), 0.0, size - 1.0)
                y = jnp.clip(w * (size / IMG_W), 0.0, size - 1.0)
                x1i = x.astype(jnp.int32)
                x1f = x1i.astype(jnp.float32)
                x2i = x1i + (x > x1f).astype(jnp.int32)
                x2f = x2i.astype(jnp.float32)
                y1i = y.astype(jnp.int32)
                y1f = y1i.astype(jnp.float32)
                y2i = y1i + (y > y1f).astype(jnp.int32)
                y2f = y2i.astype(jnp.float32)

                wts.append(((x2f - x) * (y2f - y), (x - x1f) * (y2f - y),
                            (x2f - x) * (y - y1f), (x - x1f) * (y - y1f)))

                rowbase = (lvl * B + bidx) * (HW * HW)
                r1 = rowbase + x1i * HW
                r2 = rowbase + x2i * HW
                iv = idx_refs[lvl]
                iv[pl.ds(0 * L, L)] = r1 + y1i
                iv[pl.ds(1 * L, L)] = r2 + y1i
                iv[pl.ds(2 * L, L)] = r1 + y2i
                iv[pl.ds(3 * L, L)] = r2 + y2i

            cps = {
                0: pltpu.async_copy(table_hbm.at[idx0], rows_a, sem_a),
                1: pltpu.async_copy(table_hbm.at[idx1], rows_b, sem_b),
            }
            for lvl in range(NLVL):
                rbuf = row_bufs[lvl % 2]
                cps[lvl].wait()
                w11v, w21v, w12v, w22v = wts[lvl]
                col0 = 3 + lvl * C

                def blend_p(p, _, rbuf=rbuf, w11v=w11v, w21v=w21v,
                            w12v=w12v, w22v=w22v, col0=col0):
                    pfull = jnp.full((L,), p, jnp.int32)
                    w11 = w11v.at[pfull].get(mode="promise_in_bounds")
                    w21 = w21v.at[pfull].get(mode="promise_in_bounds")
                    w12 = w12v.at[pfull].get(mode="promise_in_bounds")
                    w22 = w22v.at[pfull].get(mode="promise_in_bounds")
                    dst0 = poff + p * OUTD + col0
                    for cc in range(C // L):
                        sl = pl.ds(cc * L, L)
                        acc = (w11 * rbuf[p, sl]
                               + w21 * rbuf[L + p, sl]
                               + w12 * rbuf[2 * L + p, sl]
                               + w22 * rbuf[3 * L + p, sl])
                        outb_v[pl.ds(dst0 + cc * L, L)] = acc
                    return 0

                lax.fori_loop(0, CHUNK, blend_p, 0)
                if lvl + 2 < NLVL:
                    cps[lvl + 2] = pltpu.async_copy(
                        table_hbm.at[idx_refs[lvl + 2]], rbuf, sems[lvl % 2])

            @pl.when(ci > 0)
            def _():
                pltpu.make_async_copy(
                    outb_v.at[pl.ds(OUTB - poff, OUTB)],
                    out_hbm.at[pl.ds((base + off - CHUNK) * OUTD, OUTB)],
                    sem_out).wait()

            pltpu.async_copy(outb_v.at[pl.ds(poff, OUTB)],
                             out_hbm.at[pl.ds((base + off) * OUTD, OUTB)],
                             sem_out)
            return 0

        lax.fori_loop(0, NCHUNK, chunk_body, 0)

        last_off = (NCHUNK - 1) * CHUNK
        last_poff = ((NCHUNK - 1) % 2) * OUTB
        pltpu.make_async_copy(
            outb_v.at[pl.ds(last_poff, OUTB)],
            out_hbm.at[pl.ds((base + last_off) * OUTD, OUTB)],
            sem_out).wait()

    return k(verts_t, table)


def kernel(vertices, img_feats, proj_mat):
    del proj_mat
    table = jnp.transpose(img_feats, (0, 1, 3, 4, 2)).reshape(NLVL * B * HW * HW, C)
    verts_t = jnp.transpose(vertices.reshape(PTS, 3), (1, 0))
    out = _sc_project(verts_t, table)
    return out.reshape(B, N, OUTD)


# speedup vs baseline: 11.3919x; 1.0685x over previous
import functools

import jax
import jax.numpy as jnp
from jax import lax
from jax.experimental import pallas as pl
from jax.experimental.pallas import tpu as pltpu
from jax.experimental.pallas import tpu_sc as plsc

B, N, C = 16, 8192, 256
HW = 56
LEVEL_SIZES = (56, 28, 14, 7)
NLVL = 4
FX, FY, CX, CY = 250.0, 250.0, 112.0, 112.0
IMG_H, IMG_W = 224.0, 224.0
OUTD = 3 + NLVL * C
PTS = B * N

NC, NS, L = 2, 16, 16
NW = NC * NS
PPW = PTS // NW
CHUNK = 16
NCHUNK = PPW // CHUNK
OUTB = CHUNK * OUTD


def _sc_project(verts_t, table):
    mesh = plsc.VectorSubcoreMesh(
        core_axis_name="c", subcore_axis_name="s",
        num_cores=NC, num_subcores=NS)

    @functools.partial(
        pl.kernel,
        out_type=jax.ShapeDtypeStruct((PTS * OUTD,), jnp.float32),
        mesh=mesh,
        compiler_params=pltpu.CompilerParams(needs_layout_passes=False),
        scratch_types=[
            pltpu.VMEM((3, PPW), jnp.float32),
            pltpu.VMEM((4 * L,), jnp.int32),
            pltpu.VMEM((4 * L,), jnp.int32),
            pltpu.VMEM((4 * L,), jnp.int32),
            pltpu.VMEM((4 * L,), jnp.int32),
            pltpu.VMEM((4 * L, C), jnp.float32),
            pltpu.VMEM((4 * L, C), jnp.float32),
            pltpu.VMEM((2 * OUTB,), jnp.float32),
            pltpu.SemaphoreType.DMA,
            pltpu.SemaphoreType.DMA,
            pltpu.SemaphoreType.DMA,
        ],
    )
    def k(verts_hbm, table_hbm, out_hbm, verts_v, idx0, idx1, idx2, idx3,
          rows_a, rows_b, outb_v, sem_a, sem_b, sem_out):
        wid = lax.axis_index("s") * NC + lax.axis_index("c")
        base = wid * PPW
        bidx = base // N
        pltpu.sync_copy(verts_hbm.at[:, pl.ds(base, PPW)], verts_v)

        lane = lax.iota(jnp.int32, L)
        idx_refs = (idx0, idx1, idx2, idx3)
        row_bufs = (rows_a, rows_b)
        sems = (sem_a, sem_b)

        def chunk_body(ci, _):
            off = ci * CHUNK
            poff = lax.rem(ci, 2) * OUTB
            xv = verts_v[0, pl.ds(off, L)]
            yv = verts_v[1, pl.ds(off, L)]
            zv = verts_v[2, pl.ds(off, L)]
            h = FY * (yv / zv) + CY
            w = FX * (xv / (-zv)) + CX

            rowoff = poff + lane * OUTD
            plsc.store_scatter(outb_v, [rowoff + 0], xv)
            plsc.store_scatter(outb_v, [rowoff + 1], yv)
            plsc.store_scatter(outb_v, [rowoff + 2], zv)

            wts = []
            for lvl, size in enumerate(LEVEL_SIZES):
                x = jnp.clip(h * (size / IMG_H), 0.0, size - 1.0)
                y = jnp.clip(w * (size / IMG_W), 0.0, size - 1.0)
                x1i = x.astype(jnp.int32)
                x1f = x1i.astype(jnp.float32)
                x2i = x1i + (x > x1f).astype(jnp.int32)
                x2f = x2i.astype(jnp.float32)
                y1i = y.astype(jnp.int32)
                y1f = y1i.astype(jnp.float32)
                y2i = y1i + (y > y1f).astype(jnp.int32)
                y2f = y2i.astype(jnp.float32)

                wts.append(((x2f - x) * (y2f - y), (x - x1f) * (y2f - y),
                            (x2f - x) * (y - y1f), (x - x1f) * (y - y1f)))

                rowbase = (lvl * B + bidx) * (HW * HW)
                r1 = rowbase + x1i * HW
                r2 = rowbase + x2i * HW
                iv = idx_refs[lvl]
                iv[pl.ds(0 * L, L)] = r1 + y1i
                iv[pl.ds(1 * L, L)] = r2 + y1i
                iv[pl.ds(2 * L, L)] = r1 + y2i
                iv[pl.ds(3 * L, L)] = r2 + y2i

            cps = {
                0: pltpu.async_copy(table_hbm.at[idx0], rows_a, sem_a),
                1: pltpu.async_copy(table_hbm.at[idx1], rows_b, sem_b),
            }
            for lvl in range(NLVL):
                rbuf = row_bufs[lvl % 2]
                cps[lvl].wait()
                w11v, w21v, w12v, w22v = wts[lvl]
                col0 = 3 + lvl * C

                @plsc.parallel_loop(0, CHUNK, unroll=4)
                def blend_p(p, rbuf=rbuf, w11v=w11v, w21v=w21v,
                            w12v=w12v, w22v=w22v, col0=col0):
                    pfull = jnp.full((L,), p, jnp.int32)
                    w11 = w11v.at[pfull].get(mode="promise_in_bounds")
                    w21 = w21v.at[pfull].get(mode="promise_in_bounds")
                    w12 = w12v.at[pfull].get(mode="promise_in_bounds")
                    w22 = w22v.at[pfull].get(mode="promise_in_bounds")
                    dst0 = poff + p * OUTD + col0
                    for cc in range(C // L):
                        sl = pl.ds(cc * L, L)
                        acc = (w11 * rbuf[p, sl]
                               + w21 * rbuf[L + p, sl]
                               + w12 * rbuf[2 * L + p, sl]
                               + w22 * rbuf[3 * L + p, sl])
                        outb_v[pl.ds(dst0 + cc * L, L)] = acc
                if lvl + 2 < NLVL:
                    cps[lvl + 2] = pltpu.async_copy(
                        table_hbm.at[idx_refs[lvl + 2]], rbuf, sems[lvl % 2])

            @pl.when(ci > 0)
            def _():
                pltpu.make_async_copy(
                    outb_v.at[pl.ds(OUTB - poff, OUTB)],
                    out_hbm.at[pl.ds((base + off - CHUNK) * OUTD, OUTB)],
                    sem_out).wait()

            pltpu.async_copy(outb_v.at[pl.ds(poff, OUTB)],
                             out_hbm.at[pl.ds((base + off) * OUTD, OUTB)],
                             sem_out)
            return 0

        lax.fori_loop(0, NCHUNK, chunk_body, 0)

        last_off = (NCHUNK - 1) * CHUNK
        last_poff = ((NCHUNK - 1) % 2) * OUTB
        pltpu.make_async_copy(
            outb_v.at[pl.ds(last_poff, OUTB)],
            out_hbm.at[pl.ds((base + last_off) * OUTD, OUTB)],
            sem_out).wait()

    return k(verts_t, table)


def kernel(vertices, img_feats, proj_mat):
    del proj_mat
    table = jnp.transpose(img_feats, (0, 1, 3, 4, 2)).reshape(NLVL * B * HW * HW, C)
    verts_t = jnp.transpose(vertices.reshape(PTS, 3), (1, 0))
    out = _sc_project(verts_t, table)
    return out.reshape(B, N, OUTD)
